# baseline (speedup 1.0000x reference)
import functools

import jax
import jax.numpy as jnp
from jax import lax
from jax.experimental import pallas as pl
from jax.experimental.pallas import tpu as pltpu
from jax.experimental.pallas import tpu_sc as plsc

N = 10000
E = 320000
D = 128
DP = 144
NPAD = 10016
CH = 64
NCORES = 2
NSUB = 16
NW = NCORES * NSUB
NCH_W = 160
NCH_TOT = NCH_W * NW
EPAD = NCH_TOT * CH
RPT = NPAD // NSUB


@functools.partial(
    pl.kernel,
    out_type=jax.ShapeDtypeStruct((NCORES, NPAD, DP), jnp.float32),
    mesh=plsc.VectorSubcoreMesh(core_axis_name="c", subcore_axis_name="s"),
    scratch_types=[
        pltpu.VMEM_SHARED((NPAD, DP), jnp.float32),
        pltpu.VMEM((NCH_W, CH), jnp.int32),
        pltpu.VMEM((NCH_W, CH), jnp.int32),
        pltpu.VMEM((CH, DP), jnp.float32),
        pltpu.VMEM((CH, DP), jnp.float32),
        pltpu.SemaphoreType.DMA,
        pltpu.SemaphoreType.DMA,
    ],
    compiler_params=pltpu.CompilerParams(use_tc_tiling_on_sc=False),
)
def _sc_aggregate(xa_hbm, col_hbm, row_hbm, z_hbm, out_hbm,
                  acc_sh, col_v, row_v, g0, g1, sem0, sem1):
    c = lax.axis_index("c")
    s = lax.axis_index("s")
    wid = c * NSUB + s

    pltpu.sync_copy(z_hbm, acc_sh.at[pl.ds(s * RPT, RPT)])
    pltpu.sync_copy(col_hbm.at[pl.ds(wid * NCH_W, NCH_W)], col_v)
    pltpu.sync_copy(row_hbm.at[pl.ds(wid * NCH_W, NCH_W)], row_v)
    plsc.subcore_barrier()

    pltpu.async_copy(xa_hbm.at[col_v.at[0]], g0, sem0)
    pltpu.async_copy(xa_hbm.at[col_v.at[1]], g1, sem1)

    @pl.loop(0, NCH_W - 2, step=2)
    def _(i):
        pltpu.make_async_copy(xa_hbm.at[col_v.at[i]], g0, sem0).wait()
        pltpu.sync_copy(g0, acc_sh.at[row_v.at[i]], add=True)
        pltpu.async_copy(xa_hbm.at[col_v.at[i + 2]], g0, sem0)

        pltpu.make_async_copy(xa_hbm.at[col_v.at[i + 1]], g1, sem1).wait()
        pltpu.sync_copy(g1, acc_sh.at[row_v.at[i + 1]], add=True)
        pltpu.async_copy(xa_hbm.at[col_v.at[i + 3]], g1, sem1)

    pltpu.make_async_copy(xa_hbm.at[col_v.at[NCH_W - 2]], g0, sem0).wait()
    pltpu.sync_copy(g0, acc_sh.at[row_v.at[NCH_W - 2]], add=True)
    pltpu.make_async_copy(xa_hbm.at[col_v.at[NCH_W - 1]], g1, sem1).wait()
    pltpu.sync_copy(g1, acc_sh.at[row_v.at[NCH_W - 1]], add=True)

    plsc.subcore_barrier()
    pltpu.sync_copy(acc_sh.at[pl.ds(s * RPT, RPT)],
                    out_hbm.at[c].at[pl.ds(s * RPT, RPT)])


def _combine(p_ref, o_ref):
    p0 = p_ref[0]
    p1 = p_ref[1]
    sums = p0[:, :D] + p1[:, :D]
    cnt = p0[:, D:D + 1] + p1[:, D:D + 1]
    o_ref[...] = jnp.where(cnt > 0.0, sums / jnp.maximum(cnt, 1.0), 0.0)


def kernel(x, edge_index):
    row = edge_index[0].astype(jnp.int32)
    col = edge_index[1].astype(jnp.int32)
    pad = EPAD - E
    row_p = jnp.concatenate([row, jnp.full((pad,), N, jnp.int32)]).reshape(
        NCH_TOT, CH)
    col_p = jnp.concatenate([col, jnp.zeros((pad,), jnp.int32)]).reshape(
        NCH_TOT, CH)
    xa = (jnp.zeros((N, DP), jnp.float32)
          .at[:, :D].set(x)
          .at[:, D].set(1.0))
    zeros = jnp.zeros((RPT, DP), jnp.float32)

    partial = _sc_aggregate(xa, col_p, row_p, zeros)

    RB = 1000
    out = pl.pallas_call(
        _combine,
        out_shape=jax.ShapeDtypeStruct((N, D), jnp.float32),
        grid=(N // RB,),
        in_specs=[pl.BlockSpec((NCORES, RB, DP), lambda i: (0, i, 0))],
        out_specs=pl.BlockSpec((RB, D), lambda i: (i, 0)),
    )(partial)
    return out


# speedup vs baseline: 1.0399x; 1.0399x over previous
import functools

import jax
import jax.numpy as jnp
from jax import lax
from jax.experimental import pallas as pl
from jax.experimental.pallas import tpu as pltpu
from jax.experimental.pallas import tpu_sc as plsc

N = 10000
E = 320000
D = 128
DP = 144
NPAD = 10016
CH = 64
NCORES = 2
NSUB = 16
NW = NCORES * NSUB
NCH_W = 160
NCH_TOT = NCH_W * NW
EPAD = NCH_TOT * CH
RPT = NPAD // NSUB


@functools.partial(
    pl.kernel,
    out_type=jax.ShapeDtypeStruct((NCORES, NPAD, DP), jnp.float32),
    mesh=plsc.VectorSubcoreMesh(core_axis_name="c", subcore_axis_name="s"),
    scratch_types=[
        pltpu.VMEM_SHARED((NPAD, DP), jnp.float32),
        pltpu.VMEM((NCH_W, CH), jnp.int32),
        pltpu.VMEM((NCH_W, CH), jnp.int32),
        pltpu.VMEM((CH, DP), jnp.float32),
        pltpu.VMEM((CH, DP), jnp.float32),
        pltpu.SemaphoreType.DMA,
        pltpu.SemaphoreType.DMA,
    ],
    compiler_params=pltpu.CompilerParams(use_tc_tiling_on_sc=False),
)
def _sc_aggregate(xa_hbm, col_hbm, row_hbm, z_hbm, out_hbm,
                  acc_sh, col_v, row_v, g0, g1, sem0, sem1):
    c = lax.axis_index("c")
    s = lax.axis_index("s")
    wid = c * NSUB + s

    pltpu.sync_copy(z_hbm, acc_sh.at[pl.ds(s * RPT, RPT)])
    pltpu.sync_copy(col_hbm.at[pl.ds(wid * NCH_W, NCH_W)], col_v)
    pltpu.sync_copy(row_hbm.at[pl.ds(wid * NCH_W, NCH_W)], row_v)
    plsc.subcore_barrier()

    pltpu.async_copy(xa_hbm.at[col_v.at[0]], g0, sem0)
    pltpu.async_copy(xa_hbm.at[col_v.at[1]], g1, sem1)

    @pl.loop(0, NCH_W - 2, step=2)
    def _(i):
        pltpu.make_async_copy(xa_hbm.at[col_v.at[i]], g0, sem0).wait()
        pltpu.sync_copy(g0, acc_sh.at[row_v.at[i]], add=True)
        pltpu.async_copy(xa_hbm.at[col_v.at[i + 2]], g0, sem0)

        pltpu.make_async_copy(xa_hbm.at[col_v.at[i + 1]], g1, sem1).wait()
        pltpu.sync_copy(g1, acc_sh.at[row_v.at[i + 1]], add=True)
        pltpu.async_copy(xa_hbm.at[col_v.at[i + 3]], g1, sem1)

    pltpu.make_async_copy(xa_hbm.at[col_v.at[NCH_W - 2]], g0, sem0).wait()
    pltpu.sync_copy(g0, acc_sh.at[row_v.at[NCH_W - 2]], add=True)
    pltpu.make_async_copy(xa_hbm.at[col_v.at[NCH_W - 1]], g1, sem1).wait()
    pltpu.sync_copy(g1, acc_sh.at[row_v.at[NCH_W - 1]], add=True)

    plsc.subcore_barrier()
    pltpu.sync_copy(acc_sh.at[pl.ds(s * RPT, RPT)],
                    out_hbm.at[c].at[pl.ds(s * RPT, RPT)])


def _combine(p_ref, o_ref):
    p0 = p_ref[0]
    p1 = p_ref[1]
    sums = p0[:, :D] + p1[:, :D]
    cnt = p0[:, D:D + 1] + p1[:, D:D + 1]
    o_ref[...] = jnp.where(cnt > 0.0, sums / jnp.maximum(cnt, 1.0), 0.0)


def kernel(x, edge_index):
    row = edge_index[0].astype(jnp.int32)
    col = edge_index[1].astype(jnp.int32)
    pad = EPAD - E
    dummy_rows = N + jnp.arange(pad, dtype=jnp.int32) % (NPAD - N)
    row_p = jnp.concatenate([row, dummy_rows]).reshape(NCH_TOT, CH)
    col_p = jnp.concatenate([col, jnp.zeros((pad,), jnp.int32)]).reshape(
        NCH_TOT, CH)
    xa = (jnp.zeros((N, DP), jnp.float32)
          .at[:, :D].set(x)
          .at[:, D].set(1.0))
    zeros = jnp.zeros((RPT, DP), jnp.float32)

    partial = _sc_aggregate(xa, col_p, row_p, zeros)

    RB = 1000
    out = pl.pallas_call(
        _combine,
        out_shape=jax.ShapeDtypeStruct((N, D), jnp.float32),
        grid=(N // RB,),
        in_specs=[pl.BlockSpec((NCORES, RB, DP), lambda i: (0, i, 0))],
        out_specs=pl.BlockSpec((RB, D), lambda i: (i, 0)),
    )(partial)
    return out
